# baseline (speedup 1.0000x reference)
import functools

import jax
import jax.numpy as jnp
from jax import lax
from jax.experimental import pallas as pl
from jax.experimental.pallas import tpu as pltpu
from jax.experimental.pallas import tpu_sc as plsc

BATCH = 4096
HIDDEN = 1024
NUM_CORES = 2
NUM_SUBCORES = 16
NUM_WORKERS = NUM_CORES * NUM_SUBCORES
B_PER_W = BATCH // NUM_WORKERS
CHUNK = 32
NCHUNK = B_PER_W // CHUNK


@functools.partial(
    pl.kernel,
    mesh=plsc.VectorSubcoreMesh(core_axis_name="c", subcore_axis_name="s"),
    out_type=jax.ShapeDtypeStruct((BATCH, HIDDEN), jnp.float32),
    scratch_types=[
        pltpu.VMEM((B_PER_W,), jnp.int32),
        pltpu.VMEM((2, CHUNK, HIDDEN), jnp.float32),
        pltpu.SemaphoreType.DMA,
        pltpu.SemaphoreType.DMA,
    ],
)
def _gather_kernel(table_hbm, idx_hbm, out_hbm, idx_v, rows_v, gsem, osem):
    wid = lax.axis_index("s") * NUM_CORES + lax.axis_index("c")
    base = wid * B_PER_W
    pltpu.sync_copy(idx_hbm.at[pl.ds(base, B_PER_W)], idx_v)

    pltpu.async_copy(
        table_hbm.at[idx_v.at[pl.ds(0, CHUNK)]], rows_v.at[0], gsem
    )
    for c in range(NCHUNK):
        buf = c % 2
        pltpu.make_async_copy(
            table_hbm.at[idx_v.at[pl.ds(c * CHUNK, CHUNK)]], rows_v.at[buf], gsem
        ).wait()
        if c + 1 < NCHUNK:
            pltpu.async_copy(
                table_hbm.at[idx_v.at[pl.ds((c + 1) * CHUNK, CHUNK)]],
                rows_v.at[1 - buf],
                gsem,
            )
        if c >= 2:
            pltpu.make_async_copy(
                rows_v.at[buf], out_hbm.at[pl.ds(base + (c - 2) * CHUNK, CHUNK)], osem
            ).wait()
        pltpu.async_copy(
            rows_v.at[buf], out_hbm.at[pl.ds(base + c * CHUNK, CHUNK)], osem
        )
    for c in range(NCHUNK - 2, NCHUNK):
        pltpu.make_async_copy(
            rows_v.at[c % 2], out_hbm.at[pl.ds(base + c * CHUNK, CHUNK)], osem
        ).wait()


def kernel(labels, embedding_table):
    return _gather_kernel(embedding_table, labels.astype(jnp.int32))


# speedup vs baseline: 1.0565x; 1.0565x over previous
import functools

import jax
import jax.numpy as jnp
from jax import lax
from jax.experimental import pallas as pl
from jax.experimental.pallas import tpu as pltpu
from jax.experimental.pallas import tpu_sc as plsc

BATCH = 4096
HIDDEN = 1024
NUM_CORES = 2
NUM_SUBCORES = 16
NUM_WORKERS = NUM_CORES * NUM_SUBCORES
B_PER_W = BATCH // NUM_WORKERS
CHUNK = 16
NBUF = 7
NCHUNK = B_PER_W // CHUNK


@functools.partial(
    pl.kernel,
    mesh=plsc.VectorSubcoreMesh(core_axis_name="c", subcore_axis_name="s"),
    out_type=jax.ShapeDtypeStruct((BATCH, HIDDEN), jnp.float32),
    scratch_types=[
        pltpu.VMEM((B_PER_W,), jnp.int32),
        pltpu.VMEM((NBUF, CHUNK, HIDDEN), jnp.float32),
        pltpu.SemaphoreType.DMA,
        pltpu.SemaphoreType.DMA,
    ],
)
def _gather_kernel(table_hbm, idx_hbm, out_hbm, idx_v, rows_v, gsem, osem):
    wid = lax.axis_index("s") * NUM_CORES + lax.axis_index("c")
    base = wid * B_PER_W

    def gather(c):
        pltpu.async_copy(
            table_hbm.at[idx_v.at[pl.ds(c * CHUNK, CHUNK)]],
            rows_v.at[c % NBUF],
            gsem,
        )

    def out_copy(c):
        return pltpu.make_async_copy(
            rows_v.at[c % NBUF], out_hbm.at[pl.ds(base + c * CHUNK, CHUNK)], osem
        )

    pltpu.sync_copy(idx_hbm.at[pl.ds(base, B_PER_W)], idx_v)

    for c in range(min(NBUF - 1, NCHUNK)):
        gather(c)
    for c in range(NCHUNK):
        pltpu.make_async_copy(
            table_hbm.at[idx_v.at[pl.ds(c * CHUNK, CHUNK)]],
            rows_v.at[c % NBUF],
            gsem,
        ).wait()
        out_copy(c).start()
        nxt = c + NBUF - 1
        if nxt < NCHUNK:
            drain = nxt - NBUF
            if drain >= 0:
                out_copy(drain).wait()
            gather(nxt)
    for c in range(max(NCHUNK - NBUF, 0), NCHUNK):
        out_copy(c).wait()


def kernel(labels, embedding_table):
    return _gather_kernel(embedding_table, labels.astype(jnp.int32))


# speedup vs baseline: 1.0619x; 1.0052x over previous
import functools

import jax
import jax.numpy as jnp
from jax import lax
from jax.experimental import pallas as pl
from jax.experimental.pallas import tpu as pltpu
from jax.experimental.pallas import tpu_sc as plsc

BATCH = 4096
HIDDEN = 1024
NUM_CORES = 2
NUM_SUBCORES = 16
NUM_WORKERS = NUM_CORES * NUM_SUBCORES
B_PER_W = BATCH // NUM_WORKERS
CHUNK = 32
NBUF = 3
NCHUNK = B_PER_W // CHUNK


@functools.partial(
    pl.kernel,
    mesh=plsc.VectorSubcoreMesh(core_axis_name="c", subcore_axis_name="s"),
    out_type=jax.ShapeDtypeStruct((BATCH, HIDDEN), jnp.float32),
    scratch_types=[
        pltpu.VMEM((B_PER_W,), jnp.int32),
        pltpu.VMEM((NBUF, CHUNK, HIDDEN), jnp.float32),
        pltpu.SemaphoreType.DMA,
        pltpu.SemaphoreType.DMA,
    ],
)
def _gather_kernel(table_hbm, idx_hbm, out_hbm, idx_v, rows_v, gsem, osem):
    wid = lax.axis_index("s") * NUM_CORES + lax.axis_index("c")
    base = wid * B_PER_W

    def gather(c):
        pltpu.async_copy(
            table_hbm.at[idx_v.at[pl.ds(c * CHUNK, CHUNK)]],
            rows_v.at[c % NBUF],
            gsem,
        )

    def out_copy(c):
        return pltpu.make_async_copy(
            rows_v.at[c % NBUF], out_hbm.at[pl.ds(base + c * CHUNK, CHUNK)], osem
        )

    pltpu.sync_copy(idx_hbm.at[pl.ds(base, B_PER_W)], idx_v)

    for c in range(min(NBUF - 1, NCHUNK)):
        gather(c)
    for c in range(NCHUNK):
        pltpu.make_async_copy(
            table_hbm.at[idx_v.at[pl.ds(c * CHUNK, CHUNK)]],
            rows_v.at[c % NBUF],
            gsem,
        ).wait()
        out_copy(c).start()
        nxt = c + NBUF - 1
        if nxt < NCHUNK:
            drain = nxt - NBUF
            if drain >= 0:
                out_copy(drain).wait()
            gather(nxt)
    for c in range(max(NCHUNK - NBUF, 0), NCHUNK):
        out_copy(c).wait()


def kernel(labels, embedding_table):
    return _gather_kernel(embedding_table, labels.astype(jnp.int32))
